# baseline (speedup 1.0000x reference)
import numpy as np
import jax
import jax.numpy as jnp
from jax import lax
from jax.experimental import pallas as pl
from jax.experimental.pallas import tpu as pltpu
from jax.experimental.pallas import tpu_sc as plsc

N = 10000
E = 320000
D = 128
NCORES = 2
NSUB = 16
NW = NCORES * NSUB
CH = 128
NCH = 80
EPT = NCH * CH
EPAD = EPT * NW
NPAD = 10240
STRIPE = NPAD // NSUB
ICH = STRIPE // CH
CROWS = NPAD // D

_MESH = plsc.VectorSubcoreMesh(core_axis_name="c", subcore_axis_name="s")
_SC_PARAMS = pltpu.CompilerParams(needs_layout_passes=False)


def _sc_agg_body(y_hbm, src_hbm, dst_hbm, rowidx_hbm, z_hbm, p_hbm,
                 acc, src0, dst0, src1, dst1, rows0, rows1,
                 gs0, gs1, ss0, ss1):
    cid = lax.axis_index("c")
    sid = lax.axis_index("s")
    w = cid * NSUB + sid
    s0 = sid * STRIPE

    pltpu.sync_copy(z_hbm, rows0)
    for k in range(ICH):
        pltpu.sync_copy(rowidx_hbm.at[pl.ds(s0 + k * CH, CH)], dst0)
        pltpu.sync_copy(rows0, acc.at[dst0])
    plsc.subcore_barrier()

    e0 = w * EPT
    srcs = (src0, src1)
    dsts = (dst0, dst1)
    rows = (rows0, rows1)
    gsems = (gs0, gs1)
    ssems = (ss0, ss1)

    def load_and_gather(par, c):
        base = pl.multiple_of(e0 + c * CH, CH)
        pltpu.sync_copy(src_hbm.at[pl.ds(base, CH)], srcs[par])
        pltpu.sync_copy(dst_hbm.at[pl.ds(base, CH)], dsts[par])
        pltpu.async_copy(y_hbm.at[srcs[par]], rows[par], gsems[par])

    def wait_gather(par):
        pltpu.make_async_copy(y_hbm.at[srcs[par]], rows[par],
                              gsems[par]).wait()

    def start_scatter(par):
        pltpu.async_copy(rows[par], acc.at[dsts[par]], ssems[par], add=True)

    def wait_scatter(par):
        pltpu.make_async_copy(rows[par], acc.at[dsts[par]],
                              ssems[par]).wait()

    load_and_gather(0, 0)
    load_and_gather(1, 1)

    def step(g, carry):
        c = 2 * g
        wait_gather(0)
        start_scatter(0)
        wait_gather(1)
        start_scatter(1)
        wait_scatter(0)
        load_and_gather(0, c + 2)
        wait_scatter(1)
        load_and_gather(1, c + 3)
        return carry

    lax.fori_loop(0, NCH // 2 - 1, step, 0)
    wait_gather(0)
    start_scatter(0)
    wait_gather(1)
    start_scatter(1)
    wait_scatter(0)
    wait_scatter(1)
    plsc.subcore_barrier()

    o0 = cid * NPAD + s0
    for k in range(ICH):
        pltpu.sync_copy(rowidx_hbm.at[pl.ds(s0 + k * CH, CH)], dst0)
        pltpu.sync_copy(acc.at[dst0], rows0)
        pltpu.sync_copy(rows0, p_hbm.at[pl.ds(o0 + k * CH, CH)])


_sc_agg = pl.kernel(
    _sc_agg_body,
    out_type=jax.ShapeDtypeStruct((NCORES * NPAD, D), jnp.float32),
    mesh=_MESH,
    scratch_types=[
        pltpu.VMEM_SHARED((NPAD, D), jnp.float32),
        pltpu.VMEM((CH,), jnp.int32),
        pltpu.VMEM((CH,), jnp.int32),
        pltpu.VMEM((CH,), jnp.int32),
        pltpu.VMEM((CH,), jnp.int32),
        pltpu.VMEM((CH, D), jnp.float32),
        pltpu.VMEM((CH, D), jnp.float32),
        pltpu.SemaphoreType.DMA,
        pltpu.SemaphoreType.DMA,
        pltpu.SemaphoreType.DMA,
        pltpu.SemaphoreType.DMA,
    ],
    compiler_params=_SC_PARAMS,
)


def _sc_cnt_body(dst_hbm, rowidx_hbm, z_hbm, z1_hbm, cnt_hbm,
                 cacc, dst_v, idx_v, rows_v, hist):
    cid = lax.axis_index("c")
    sid = lax.axis_index("s")
    w = cid * NSUB + sid

    pltpu.sync_copy(z_hbm, rows_v)
    pltpu.sync_copy(z1_hbm, hist)
    pltpu.sync_copy(rowidx_hbm.at[pl.ds(0, CH)], idx_v)

    @pl.when(sid == 0)
    def _():
        pltpu.sync_copy(rows_v, cacc.at[idx_v])
    plsc.subcore_barrier()

    vone = jnp.ones((16,), jnp.float32)

    def step(c, carry):
        base = pl.multiple_of(w * EPT + c * CH, CH)
        pltpu.sync_copy(dst_hbm.at[pl.ds(base, CH)], dst_v)
        for j in range(CH // 16):
            dvec = dst_v[pl.ds(j * 16, 16)]
            plsc.addupdate_scatter(hist, [dvec], vone)
        return carry

    lax.fori_loop(0, NCH, step, 0)

    def pack(r, carry):
        for j in range(D // 16):
            rows_v[r, pl.ds(j * 16, 16)] = hist[pl.ds(r * D + j * 16, 16)]
        return carry

    lax.fori_loop(0, CROWS, pack, 0)
    pltpu.sync_copy(rows_v, cacc.at[idx_v], add=True)
    plsc.subcore_barrier()

    @pl.when(sid == 0)
    def _():
        pltpu.sync_copy(cacc.at[idx_v], rows_v)
        pltpu.sync_copy(rows_v, cnt_hbm.at[pl.ds(cid * CH, CH)])


_sc_cnt = pl.kernel(
    _sc_cnt_body,
    out_type=jax.ShapeDtypeStruct((NCORES * CH, D), jnp.float32),
    mesh=_MESH,
    scratch_types=[
        pltpu.VMEM_SHARED((CH, D), jnp.float32),
        pltpu.VMEM((CH,), jnp.int32),
        pltpu.VMEM((CH,), jnp.int32),
        pltpu.VMEM((CH, D), jnp.float32),
        pltpu.VMEM((NPAD,), jnp.float32),
    ],
    compiler_params=_SC_PARAMS,
)

RB = 1280
_GRID = (NPAD // RB,)


def _dotT(a, w):
    return lax.dot_general(a, w, (((1,), (1,)), ((), ())),
                           precision=lax.Precision.HIGHEST)


def _cnt_col(c0, c1, eq, msk):
    cb = c0 + c1
    expanded = lax.dot_general(eq, cb, (((1,), (0,)), ((), ())),
                               precision=lax.Precision.HIGHEST)
    col = jnp.sum(expanded * msk, axis=1, keepdims=True)
    return jnp.maximum(col, 1.0)


def _tc_prep_body(x_ref, wl_ref, wr_ref, b_ref, y_ref, r_ref):
    xb = x_ref[...]
    y_ref[...] = _dotT(xb, wl_ref[...])
    r_ref[...] = _dotT(xb, wr_ref[...]) + b_ref[...]


def _tc_mid_body(p0_ref, p1_ref, c0_ref, c1_ref, r1_ref, eq_ref, msk_ref,
                 wl_ref, wr_ref, b_ref, y2_ref, r2_ref):
    cnt = _cnt_col(c0_ref[...], c1_ref[...], eq_ref[...], msk_ref[...])
    h = jnp.maximum((p0_ref[...] + p1_ref[...]) / cnt + r1_ref[...], 0.0)
    y2_ref[...] = _dotT(h, wl_ref[...])
    r2_ref[...] = _dotT(h, wr_ref[...]) + b_ref[...]


def _tc_fin_body(q0_ref, q1_ref, c0_ref, c1_ref, r2_ref, eq_ref, msk_ref,
                 out_ref):
    cnt = _cnt_col(c0_ref[...], c1_ref[...], eq_ref[...], msk_ref[...])
    out_ref[...] = (q0_ref[...] + q1_ref[...]) / cnt + r2_ref[...]


_row_spec = pl.BlockSpec((RB, D), lambda i: (i, 0))
_cnt_spec = pl.BlockSpec((CROWS, D), lambda i: (0, 0))
_eq_spec = pl.BlockSpec((RB, CROWS), lambda i: (i, 0))
_w_spec = pl.BlockSpec((D, D), lambda i: (0, 0))
_b_spec = pl.BlockSpec((1, D), lambda i: (0, 0))

_tc_prep = pl.pallas_call(
    _tc_prep_body, grid=_GRID,
    in_specs=[_row_spec, _w_spec, _w_spec, _b_spec],
    out_specs=[_row_spec, _row_spec],
    out_shape=[jax.ShapeDtypeStruct((NPAD, D), jnp.float32)] * 2,
)

_tc_mid = pl.pallas_call(
    _tc_mid_body, grid=_GRID,
    in_specs=[_row_spec, _row_spec, _cnt_spec, _cnt_spec, _row_spec,
              _eq_spec, _row_spec, _w_spec, _w_spec, _b_spec],
    out_specs=[_row_spec, _row_spec],
    out_shape=[jax.ShapeDtypeStruct((NPAD, D), jnp.float32)] * 2,
)

_tc_fin = pl.pallas_call(
    _tc_fin_body, grid=_GRID,
    in_specs=[_row_spec, _row_spec, _cnt_spec, _cnt_spec, _row_spec,
              _eq_spec, _row_spec],
    out_specs=_row_spec,
    out_shape=jax.ShapeDtypeStruct((NPAD, D), jnp.float32),
)

_EQ = np.asarray(
    np.arange(NPAD)[:, None] // D == np.arange(CROWS)[None, :],
    dtype=np.float32)
_MSK = np.asarray(
    np.arange(RB)[:, None] % D == np.arange(D)[None, :], dtype=np.float32)


def kernel(x, edge_index, W1l, b1, W1r, W2l, b2, W2r):
    src = edge_index[0].astype(jnp.int32)
    dst = edge_index[1].astype(jnp.int32)
    pad = EPAD - E
    srcp = jnp.concatenate([src, jnp.zeros((pad,), jnp.int32)])
    dstp = jnp.concatenate([dst, jnp.full((pad,), N, jnp.int32)])
    zacc = jnp.zeros((CH, D), jnp.float32)
    zflat = jnp.zeros((NPAD,), jnp.float32)
    rowidx = jnp.arange(NPAD, dtype=jnp.int32)
    xp = jnp.concatenate([x, jnp.zeros((NPAD - N, D), jnp.float32)])

    cnt = _sc_cnt(dstp, rowidx, zacc, zflat)
    c0, c1 = cnt[:CROWS], cnt[CH:CH + CROWS]
    y1, r1 = _tc_prep(xp, W1l, W1r, b1.reshape(1, D))
    p = _sc_agg(y1, srcp, dstp, rowidx, zacc).reshape(NCORES, NPAD, D)
    eq = jnp.asarray(_EQ)
    msk = jnp.asarray(_MSK)
    y2, r2 = _tc_mid(p[0], p[1], c0, c1, r1, eq, msk,
                     W2l, W2r, b2.reshape(1, D))
    q = _sc_agg(y2, srcp, dstp, rowidx, zacc).reshape(NCORES, NPAD, D)
    return _tc_fin(q[0], q[1], c0, c1, r2, eq, msk)[:N]


# speedup vs baseline: 2.6266x; 2.6266x over previous
import numpy as np
import jax
import jax.numpy as jnp
from jax import lax
from jax.experimental import pallas as pl
from jax.experimental.pallas import tpu as pltpu
from jax.experimental.pallas import tpu_sc as plsc

N = 10000
E = 320000
D = 128
NCORES = 2
NSUB = 16
NW = NCORES * NSUB
CH = 128
NCH = 80
EPT = NCH * CH
EPAD = EPT * NW
NPAD = 10240
STRIPE = NPAD // NSUB
ICH = STRIPE // CH
CROWS = NPAD // D

_MESH = plsc.VectorSubcoreMesh(core_axis_name="c", subcore_axis_name="s")
_SC_PARAMS = pltpu.CompilerParams(needs_layout_passes=False)


def _make_sc_agg(with_counts):
    out_type = [jax.ShapeDtypeStruct((NCORES * NPAD, D), jnp.float32)]
    scratch = [
        pltpu.VMEM_SHARED((NPAD, D), jnp.float32),
        pltpu.VMEM((CH,), jnp.int32),
        pltpu.VMEM((CH,), jnp.int32),
        pltpu.VMEM((CH,), jnp.int32),
        pltpu.VMEM((CH,), jnp.int32),
        pltpu.VMEM((CH, D), jnp.float32),
        pltpu.VMEM((CH, D), jnp.float32),
        pltpu.SemaphoreType.DMA,
        pltpu.SemaphoreType.DMA,
        pltpu.SemaphoreType.DMA,
        pltpu.SemaphoreType.DMA,
    ]
    if with_counts:
        out_type.append(jax.ShapeDtypeStruct((NCORES * CH, D), jnp.float32))
        scratch += [
            pltpu.VMEM_SHARED((CH, D), jnp.float32),
            pltpu.VMEM((NPAD,), jnp.float32),
        ]

    def body(y_hbm, src_hbm, dst_hbm, rowidx_hbm, z_hbm, *rest):
        if with_counts:
            (z1_hbm, p_hbm, cnt_hbm, acc, src0, dst0, src1, dst1,
             rows0, rows1, gs0, gs1, ss0, ss1, cacc, hist) = rest
        else:
            (p_hbm, acc, src0, dst0, src1, dst1,
             rows0, rows1, gs0, gs1, ss0, ss1) = rest
        cid = lax.axis_index("c")
        sid = lax.axis_index("s")
        w = cid * NSUB + sid
        s0 = sid * STRIPE

        pltpu.sync_copy(z_hbm, rows0)
        for k in range(ICH):
            pltpu.sync_copy(rowidx_hbm.at[pl.ds(s0 + k * CH, CH)], dst0)
            pltpu.sync_copy(rows0, acc.at[dst0])
        if with_counts:
            pltpu.sync_copy(z1_hbm, hist)
            pltpu.sync_copy(rowidx_hbm.at[pl.ds(0, CH)], dst0)

            @pl.when(sid == 0)
            def _():
                pltpu.sync_copy(rows0, cacc.at[dst0])
        plsc.subcore_barrier()

        e0 = w * EPT
        srcs = (src0, src1)
        dsts = (dst0, dst1)
        rows = (rows0, rows1)
        gsems = (gs0, gs1)
        ssems = (ss0, ss1)
        vone = jnp.ones((16,), jnp.float32)

        def load_and_gather(par, c):
            base = pl.multiple_of(e0 + c * CH, CH)
            pltpu.sync_copy(src_hbm.at[pl.ds(base, CH)], srcs[par])
            pltpu.sync_copy(dst_hbm.at[pl.ds(base, CH)], dsts[par])
            pltpu.async_copy(y_hbm.at[srcs[par]], rows[par], gsems[par])
            if with_counts:
                for j in range(CH // 16):
                    dvec = dsts[par][pl.ds(j * 16, 16)]
                    plsc.addupdate_scatter(hist, [dvec], vone)

        def wait_gather(par):
            pltpu.make_async_copy(y_hbm.at[srcs[par]], rows[par],
                                  gsems[par]).wait()

        def start_scatter(par):
            pltpu.async_copy(rows[par], acc.at[dsts[par]], ssems[par],
                             add=True)

        def wait_scatter(par):
            pltpu.make_async_copy(rows[par], acc.at[dsts[par]],
                                  ssems[par]).wait()

        load_and_gather(0, 0)
        load_and_gather(1, 1)

        def step(g, carry):
            c = 2 * g
            wait_gather(0)
            start_scatter(0)
            wait_gather(1)
            start_scatter(1)
            wait_scatter(0)
            load_and_gather(0, c + 2)
            wait_scatter(1)
            load_and_gather(1, c + 3)
            return carry

        lax.fori_loop(0, NCH // 2 - 1, step, 0)
        wait_gather(0)
        start_scatter(0)
        wait_gather(1)
        start_scatter(1)
        wait_scatter(0)
        wait_scatter(1)
        if with_counts:
            pltpu.sync_copy(z_hbm, rows0)

            def pack(r, carry):
                for j in range(D // 16):
                    rows0[r, pl.ds(j * 16, 16)] = (
                        hist[pl.ds(r * D + j * 16, 16)])
                return carry

            lax.fori_loop(0, CROWS, pack, 0)
            pltpu.sync_copy(rowidx_hbm.at[pl.ds(0, CH)], dst0)
            pltpu.sync_copy(rows0, cacc.at[dst0], add=True)
        plsc.subcore_barrier()

        o0 = cid * NPAD + s0
        for k in range(ICH):
            pltpu.sync_copy(rowidx_hbm.at[pl.ds(s0 + k * CH, CH)], dst0)
            pltpu.sync_copy(acc.at[dst0], rows0)
            pltpu.sync_copy(rows0, p_hbm.at[pl.ds(o0 + k * CH, CH)])
        if with_counts:
            @pl.when(sid == 0)
            def _():
                pltpu.sync_copy(rowidx_hbm.at[pl.ds(0, CH)], dst0)
                pltpu.sync_copy(cacc.at[dst0], rows0)
                pltpu.sync_copy(rows0, cnt_hbm.at[pl.ds(cid * CH, CH)])

    return pl.kernel(
        body,
        out_type=tuple(out_type) if with_counts else out_type[0],
        mesh=_MESH,
        scratch_types=scratch,
        compiler_params=_SC_PARAMS,
    )


_sc_agg_counts = _make_sc_agg(True)
_sc_agg = _make_sc_agg(False)

RB = 1280
_GRID = (NPAD // RB,)


def _dotT(a, w):
    return lax.dot_general(a, w, (((1,), (1,)), ((), ())),
                           precision=lax.Precision.HIGHEST)


def _cnt_col(c0, c1, eq, msk):
    cb = c0 + c1
    expanded = lax.dot_general(eq, cb, (((1,), (0,)), ((), ())),
                               precision=lax.Precision.HIGHEST)
    col = jnp.sum(expanded * msk, axis=1, keepdims=True)
    return jnp.maximum(col, 1.0)


def _tc_prep_body(x_ref, wl_ref, wr_ref, b_ref, y_ref, r_ref):
    xb = x_ref[...]
    y_ref[...] = _dotT(xb, wl_ref[...])
    r_ref[...] = _dotT(xb, wr_ref[...]) + b_ref[...]


def _tc_mid_body(p0_ref, p1_ref, c0_ref, c1_ref, r1_ref, eq_ref, msk_ref,
                 wl_ref, wr_ref, b_ref, y2_ref, r2_ref):
    cnt = _cnt_col(c0_ref[...], c1_ref[...], eq_ref[...], msk_ref[...])
    h = jnp.maximum((p0_ref[...] + p1_ref[...]) / cnt + r1_ref[...], 0.0)
    y2_ref[...] = _dotT(h, wl_ref[...])
    r2_ref[...] = _dotT(h, wr_ref[...]) + b_ref[...]


def _tc_fin_body(q0_ref, q1_ref, c0_ref, c1_ref, r2_ref, eq_ref, msk_ref,
                 out_ref):
    cnt = _cnt_col(c0_ref[...], c1_ref[...], eq_ref[...], msk_ref[...])
    out_ref[...] = (q0_ref[...] + q1_ref[...]) / cnt + r2_ref[...]


_row_spec = pl.BlockSpec((RB, D), lambda i: (i, 0))
_cnt_spec = pl.BlockSpec((CROWS, D), lambda i: (0, 0))
_eq_spec = pl.BlockSpec((RB, CROWS), lambda i: (i, 0))
_w_spec = pl.BlockSpec((D, D), lambda i: (0, 0))
_b_spec = pl.BlockSpec((1, D), lambda i: (0, 0))

_tc_prep = pl.pallas_call(
    _tc_prep_body, grid=_GRID,
    in_specs=[_row_spec, _w_spec, _w_spec, _b_spec],
    out_specs=[_row_spec, _row_spec],
    out_shape=[jax.ShapeDtypeStruct((NPAD, D), jnp.float32)] * 2,
)

_tc_mid = pl.pallas_call(
    _tc_mid_body, grid=_GRID,
    in_specs=[_row_spec, _row_spec, _cnt_spec, _cnt_spec, _row_spec,
              _eq_spec, _row_spec, _w_spec, _w_spec, _b_spec],
    out_specs=[_row_spec, _row_spec],
    out_shape=[jax.ShapeDtypeStruct((NPAD, D), jnp.float32)] * 2,
)

_tc_fin = pl.pallas_call(
    _tc_fin_body, grid=_GRID,
    in_specs=[_row_spec, _row_spec, _cnt_spec, _cnt_spec, _row_spec,
              _eq_spec, _row_spec],
    out_specs=_row_spec,
    out_shape=jax.ShapeDtypeStruct((NPAD, D), jnp.float32),
)

_EQ = np.asarray(
    np.arange(NPAD)[:, None] // D == np.arange(CROWS)[None, :],
    dtype=np.float32)
_MSK = np.asarray(
    np.arange(RB)[:, None] % D == np.arange(D)[None, :], dtype=np.float32)


def kernel(x, edge_index, W1l, b1, W1r, W2l, b2, W2r):
    src = edge_index[0].astype(jnp.int32)
    dst = edge_index[1].astype(jnp.int32)
    pad = EPAD - E
    pad_src = jnp.asarray(np.arange(pad) % N, dtype=jnp.int32)
    pad_dst = jnp.asarray(N + np.arange(pad) % (NPAD - N), dtype=jnp.int32)
    srcp = jnp.concatenate([src, pad_src])
    dstp = jnp.concatenate([dst, pad_dst])
    zacc = jnp.zeros((CH, D), jnp.float32)
    zflat = jnp.zeros((NPAD,), jnp.float32)
    rowidx = jnp.arange(NPAD, dtype=jnp.int32)
    xp = jnp.concatenate([x, jnp.zeros((NPAD - N, D), jnp.float32)])

    y1, r1 = _tc_prep(xp, W1l, W1r, b1.reshape(1, D))
    p, cnt = _sc_agg_counts(y1, srcp, dstp, rowidx, zacc, zflat)
    p = p.reshape(NCORES, NPAD, D)
    c0, c1 = cnt[:CROWS], cnt[CH:CH + CROWS]
    eq = jnp.asarray(_EQ)
    msk = jnp.asarray(_MSK)
    y2, r2 = _tc_mid(p[0], p[1], c0, c1, r1, eq, msk,
                     W2l, W2r, b2.reshape(1, D))
    q = _sc_agg(y2, srcp, dstp, rowidx, zacc).reshape(NCORES, NPAD, D)
    return _tc_fin(q[0], q[1], c0, c1, r2, eq, msk)[:N]
